# baseline (speedup 1.0000x reference)
import jax
import jax.numpy as jnp
from jax.experimental import pallas as pl

_N = 100000
_D = 128
_K = 16
_BLOCK = 4000


def _body(x_ref, ct_ref, o_ref):
    xb = x_ref[...]
    ct = ct_ref[...]
    x2 = jnp.sum(xb * xb, axis=1, keepdims=True)
    c2 = jnp.sum(ct * ct, axis=0)[None, :]
    dist = x2 + c2 - 2.0 * jnp.dot(
        xb, ct, preferred_element_type=jnp.float32)
    dist = jnp.maximum(dist, 0.0)
    q = 1.0 / (1.0 + dist)
    o_ref[...] = q * (1.0 / jnp.sum(q, axis=1, keepdims=True))


def kernel(x, centers):
    n, d = x.shape
    k = centers.shape[0]
    ct = centers.T
    grid = (n // _BLOCK,)
    return pl.pallas_call(
        _body,
        grid=grid,
        in_specs=[
            pl.BlockSpec((_BLOCK, d), lambda i: (i, 0)),
            pl.BlockSpec((d, k), lambda i: (0, 0)),
        ],
        out_specs=pl.BlockSpec((_BLOCK, k), lambda i: (i, 0)),
        out_shape=jax.ShapeDtypeStruct((n, k), jnp.float32),
    )(x, ct)
